# baseline (speedup 1.0000x reference)
import functools

import jax
import jax.numpy as jnp
from jax import lax
from jax.experimental import pallas as pl
from jax.experimental.pallas import tpu as pltpu
from jax.experimental.pallas import tpu_sc as plsc

NUM_CORES = 2
NUM_SUBCORES = 16
NUM_WORKERS = NUM_CORES * NUM_SUBCORES


def _make_lookup(B, V, D, chunk):
    b_per_w = B // NUM_WORKERS
    n_chunks = b_per_w // chunk
    assert b_per_w % chunk == 0 and n_chunks % 2 == 0 and B % NUM_WORKERS == 0

    mesh = plsc.VectorSubcoreMesh(
        core_axis_name="c",
        subcore_axis_name="s",
        num_cores=NUM_CORES,
        num_subcores=NUM_SUBCORES,
    )

    @functools.partial(
        pl.kernel,
        out_type=jax.ShapeDtypeStruct((B, D), jnp.float32),
        mesh=mesh,
        compiler_params=pltpu.CompilerParams(use_tc_tiling_on_sc=False),
        scratch_types=[
            pltpu.VMEM((b_per_w,), jnp.int32),
            pltpu.VMEM((chunk, D), jnp.float32),
            pltpu.VMEM((chunk, D), jnp.float32),
            pltpu.SemaphoreType.DMA,
            pltpu.SemaphoreType.DMA,
            pltpu.SemaphoreType.DMA,
            pltpu.SemaphoreType.DMA,
        ],
    )
    def lookup(idx_hbm, w_hbm, out_hbm, idx_v, buf0, buf1, gs0, gs1, ws0, ws1):
        wid = lax.axis_index("s") * NUM_CORES + lax.axis_index("c")
        base = wid * b_per_w

        pltpu.sync_copy(idx_hbm.at[pl.ds(base, b_per_w)], idx_v)

        def gather(g, buf, sem):
            return pltpu.async_copy(
                w_hbm.at[idx_v.at[pl.ds(g * chunk, chunk)]], buf, sem
            )

        def write(g, buf, sem):
            return pltpu.async_copy(
                buf, out_hbm.at[pl.ds(base + g * chunk, chunk)], sem
            )

        g0 = gather(0, buf0, gs0)
        g1 = gather(1, buf1, gs1)
        g0.wait()
        write(0, buf0, ws0)
        g1.wait()
        write(1, buf1, ws1)

        @pl.loop(2, n_chunks, step=2)
        def _(g):
            pltpu.make_async_copy(
                buf0, out_hbm.at[pl.ds(base + (g - 2) * chunk, chunk)], ws0
            ).wait()
            ga = gather(g, buf0, gs0)
            pltpu.make_async_copy(
                buf1, out_hbm.at[pl.ds(base + (g - 1) * chunk, chunk)], ws1
            ).wait()
            gb = gather(g + 1, buf1, gs1)
            ga.wait()
            write(g, buf0, ws0)
            gb.wait()
            write(g + 1, buf1, ws1)

        pltpu.make_async_copy(
            buf0, out_hbm.at[pl.ds(base + (n_chunks - 2) * chunk, chunk)], ws0
        ).wait()
        pltpu.make_async_copy(
            buf1, out_hbm.at[pl.ds(base + (n_chunks - 1) * chunk, chunk)], ws1
        ).wait()

    return lookup


def kernel(x, W):
    B, H = x.shape
    V, D = W.shape
    flat = x.reshape(-1).astype(jnp.int32)
    out = _make_lookup(B * H, V, D, chunk=64)(flat, W)
    return out.reshape(B, H, D)


# speedup vs baseline: 1.0703x; 1.0703x over previous
import functools

import jax
import jax.numpy as jnp
from jax import lax
from jax.experimental import pallas as pl
from jax.experimental.pallas import tpu as pltpu
from jax.experimental.pallas import tpu_sc as plsc

NUM_CORES = 2
NUM_SUBCORES = 16
NUM_WORKERS = NUM_CORES * NUM_SUBCORES


def _make_lookup(B, V_pad, D, chunk):
    b_per_w = B // NUM_WORKERS
    n_chunks = b_per_w // chunk
    rows_per_tile = V_pad // NUM_SUBCORES
    assert b_per_w % chunk == 0 and n_chunks % 2 == 0 and B % NUM_WORKERS == 0
    assert V_pad % NUM_SUBCORES == 0

    mesh = plsc.VectorSubcoreMesh(
        core_axis_name="c",
        subcore_axis_name="s",
        num_cores=NUM_CORES,
        num_subcores=NUM_SUBCORES,
    )

    @functools.partial(
        pl.kernel,
        out_type=jax.ShapeDtypeStruct((B, D), jnp.float32),
        mesh=mesh,
        compiler_params=pltpu.CompilerParams(use_tc_tiling_on_sc=False),
        scratch_types=[
            pltpu.VMEM((b_per_w,), jnp.int32),
            pltpu.VMEM((chunk, D), jnp.float32),
            pltpu.VMEM((chunk, D), jnp.float32),
            pltpu.VMEM_SHARED((V_pad, D), jnp.float32),
            pltpu.SemaphoreType.DMA,
            pltpu.SemaphoreType.DMA,
            pltpu.SemaphoreType.DMA,
            pltpu.SemaphoreType.DMA,
        ],
    )
    def lookup(idx_hbm, w_hbm, out_hbm, idx_v, buf0, buf1, table, gs0, gs1, ws0, ws1):
        sid = lax.axis_index("s")
        wid = sid * NUM_CORES + lax.axis_index("c")
        base = wid * b_per_w

        stripe = pl.ds(sid * rows_per_tile, rows_per_tile)
        pltpu.sync_copy(w_hbm.at[stripe], table.at[stripe])

        pltpu.sync_copy(idx_hbm.at[pl.ds(base, b_per_w)], idx_v)
        plsc.subcore_barrier()

        def gather(g, buf, sem):
            return pltpu.async_copy(
                table.at[idx_v.at[pl.ds(g * chunk, chunk)]], buf, sem
            )

        def write(g, buf, sem):
            return pltpu.async_copy(
                buf, out_hbm.at[pl.ds(base + g * chunk, chunk)], sem
            )

        g0 = gather(0, buf0, gs0)
        g1 = gather(1, buf1, gs1)
        g0.wait()
        write(0, buf0, ws0)
        g1.wait()
        write(1, buf1, ws1)

        @pl.loop(2, n_chunks, step=2)
        def _(g):
            pltpu.make_async_copy(
                buf0, out_hbm.at[pl.ds(base + (g - 2) * chunk, chunk)], ws0
            ).wait()
            ga = gather(g, buf0, gs0)
            pltpu.make_async_copy(
                buf1, out_hbm.at[pl.ds(base + (g - 1) * chunk, chunk)], ws1
            ).wait()
            gb = gather(g + 1, buf1, gs1)
            ga.wait()
            write(g, buf0, ws0)
            gb.wait()
            write(g + 1, buf1, ws1)

        pltpu.make_async_copy(
            buf0, out_hbm.at[pl.ds(base + (n_chunks - 2) * chunk, chunk)], ws0
        ).wait()
        pltpu.make_async_copy(
            buf1, out_hbm.at[pl.ds(base + (n_chunks - 1) * chunk, chunk)], ws1
        ).wait()

    return lookup


def kernel(x, W):
    B, H = x.shape
    V, D = W.shape
    V_pad = -(-V // NUM_SUBCORES) * NUM_SUBCORES
    if V_pad != V:
        W = jnp.pad(W, ((0, V_pad - V), (0, 0)))
    flat = x.reshape(-1).astype(jnp.int32)
    out = _make_lookup(B * H, V_pad, D, chunk=32)(flat, W)
    return out.reshape(B, H, D)


# speedup vs baseline: 1.2125x; 1.1328x over previous
import functools

import jax
import jax.numpy as jnp
from jax import lax
from jax.experimental import pallas as pl
from jax.experimental.pallas import tpu as pltpu
from jax.experimental.pallas import tpu_sc as plsc

NUM_CORES = 2
NUM_SUBCORES = 16
NUM_WORKERS = NUM_CORES * NUM_SUBCORES


def _make_lookup(B, H, D, D_pad):
    b_per_w = B // NUM_WORKERS
    assert B % NUM_WORKERS == 0 and b_per_w % 2 == 0
    n_sub = D_pad // 128
    n_full = D // 128
    Hp = -(-H // 8) * 8
    rows_per_b = Hp * n_sub

    mesh = plsc.VectorSubcoreMesh(
        core_axis_name="c",
        subcore_axis_name="s",
        num_cores=NUM_CORES,
        num_subcores=NUM_SUBCORES,
    )

    @functools.partial(
        pl.kernel,
        out_type=[
            jax.ShapeDtypeStruct((B, H, D), jnp.float32),
            jax.ShapeDtypeStruct((B, H, 128), jnp.float32),
        ],
        mesh=mesh,
        compiler_params=pltpu.CompilerParams(use_tc_tiling_on_sc=True),
        scratch_types=[
            pltpu.VMEM((b_per_w * rows_per_b,), jnp.int32),
            pltpu.VMEM((H, D), jnp.float32),
            pltpu.VMEM((H, D), jnp.float32),
            pltpu.VMEM((H, 128), jnp.float32),
            pltpu.VMEM((H, 128), jnp.float32),
            pltpu.SemaphoreType.DMA,
            pltpu.SemaphoreType.DMA,
            pltpu.SemaphoreType.DMA,
            pltpu.SemaphoreType.DMA,
            pltpu.SemaphoreType.DMA,
            pltpu.SemaphoreType.DMA,
        ],
    )
    def lookup(idx_hbm, w_hbm, out_hbm, aux_hbm, idx_v, wb0, wb1, tb0, tb1,
               gs0, gs1, ws0, ws1, vs0, vs1):
        sid = lax.axis_index("s")
        wid = sid * NUM_CORES + lax.axis_index("c")
        base_b = wid * b_per_w

        pltpu.sync_copy(
            idx_hbm.at[pl.ds(base_b * rows_per_b, b_per_w * rows_per_b)], idx_v
        )

        def gathers(g, wb, tb, sem):
            for j in range(n_full):
                pltpu.async_copy(
                    w_hbm.at[idx_v.at[pl.ds(g * rows_per_b + j * Hp, H)]],
                    wb.at[:, pl.ds(128 * j, 128)],
                    sem,
                )
            pltpu.async_copy(
                w_hbm.at[idx_v.at[pl.ds(g * rows_per_b + n_full * Hp, H)]],
                tb,
                sem,
            )

        def wait_gathers(wb, tb, sem):
            for j in range(n_full):
                pltpu.make_async_copy(
                    w_hbm.at[idx_v.at[pl.ds(0, H)]],
                    wb.at[:, pl.ds(128 * j, 128)],
                    sem,
                ).wait()
            pltpu.make_async_copy(
                w_hbm.at[idx_v.at[pl.ds(0, H)]], tb, sem
            ).wait()

        def write(g, wb, tb, sem, sem2):
            pltpu.async_copy(wb, out_hbm.at[base_b + g], sem)
            pltpu.async_copy(tb, aux_hbm.at[base_b + g], sem2)

        def wait_write(g, wb, tb, sem, sem2):
            pltpu.make_async_copy(wb, out_hbm.at[base_b + g], sem).wait()
            pltpu.make_async_copy(tb, aux_hbm.at[base_b + g], sem2).wait()

        gathers(0, wb0, tb0, gs0)
        gathers(1, wb1, tb1, gs1)

        @pl.loop(0, b_per_w - 2, step=2)
        def _(g):
            wait_gathers(wb0, tb0, gs0)
            write(g, wb0, tb0, ws0, vs0)
            wait_gathers(wb1, tb1, gs1)
            write(g + 1, wb1, tb1, ws1, vs1)
            wait_write(g, wb0, tb0, ws0, vs0)
            gathers(g + 2, wb0, tb0, gs0)
            wait_write(g + 1, wb1, tb1, ws1, vs1)
            gathers(g + 3, wb1, tb1, gs1)

        g = b_per_w - 2
        wait_gathers(wb0, tb0, gs0)
        write(g, wb0, tb0, ws0, vs0)
        wait_gathers(wb1, tb1, gs1)
        write(g + 1, wb1, tb1, ws1, vs1)
        wait_write(g, wb0, tb0, ws0, vs0)
        wait_write(g + 1, wb1, tb1, ws1, vs1)

    return lookup


def _tail_fix(main, aux, n_full):
    B, H, D = main.shape
    bb = 256

    def body(_, aux_ref, out_ref):
        out_ref[...] = aux_ref[...]

    return pl.pallas_call(
        body,
        grid=(B // bb,),
        in_specs=[
            pl.BlockSpec(memory_space=pl.ANY),
            pl.BlockSpec((bb, H, 128), lambda i: (i, 0, 0)),
        ],
        out_specs=pl.BlockSpec((bb, H, 128), lambda i: (i, 0, n_full)),
        out_shape=jax.ShapeDtypeStruct((B, H, D), jnp.float32),
        input_output_aliases={0: 0},
    )(main, aux)


def kernel(x, W):
    B, H = x.shape
    V, D = W.shape
    D_pad = -(-D // 128) * 128
    n_sub = D_pad // 128
    Hp = -(-H // 8) * 8
    w_sub = jnp.pad(W, ((0, 0), (0, D_pad - D))).reshape(V * n_sub, 128)
    idx8 = jnp.pad(
        x.astype(jnp.int32)[:, None, :] * n_sub
        + jnp.arange(n_sub, dtype=jnp.int32)[None, :, None],
        ((0, 0), (0, 0), (0, Hp - H)),
    ).reshape(-1)
    main, aux = _make_lookup(B, H, D, D_pad)(idx8, w_sub)
    if D % 128 == 0:
        return main
    return _tail_fix(main, aux, D // 128)
